# baseline (speedup 1.0000x reference)
import functools
import jax
import jax.numpy as jnp
from jax import lax
from jax.experimental import pallas as pl
from jax.experimental.pallas import tpu as pltpu
from jax.experimental.pallas import tpu_sc as plsc

NUM_ROWS = 128
ROW_LEN = 100000
LANES = 16
NUM_CORES = 2
NUM_SUBCORES = 16
NUM_WORKERS = NUM_CORES * NUM_SUBCORES
ROWS_PER_WORKER = NUM_ROWS // NUM_WORKERS
VECS_PER_ROW = ROW_LEN // LANES


def _body(logits_hbm, out_hbm, buf, res_ref):
    c = lax.axis_index("c")
    s = lax.axis_index("s")
    wid = c * NUM_SUBCORES + s
    row0 = wid * ROWS_PER_WORKER

    lane_iota = lax.iota(jnp.int32, LANES)
    res_vec = jnp.zeros((LANES,), jnp.int32)

    for r in range(ROWS_PER_WORKER):
        pltpu.sync_copy(logits_hbm.at[row0 + r], buf)

        m0 = jnp.full((LANES,), -jnp.inf, jnp.float32)
        mi0 = jnp.zeros((LANES,), jnp.int32)

        def scan_body(i, carry):
            m, mi = carry
            v = buf[pl.ds(i * LANES, LANES)]
            gt = v > m
            m = jnp.maximum(m, v)
            mi = jnp.where(gt, jnp.full((LANES,), i, jnp.int32), mi)
            return (m, mi)

        m, mi = lax.fori_loop(0, VECS_PER_ROW, scan_body, (m0, mi0))

        full_idx = mi * LANES + lane_iota
        bv = m[0]
        bi = full_idx[0]
        for l in range(1, LANES):
            v = m[l]
            fi = full_idx[l]
            better = (v > bv) | ((v == bv) & (fi < bi))
            bv = jnp.where(better, v, bv)
            bi = jnp.where(better, fi, bi)
        res_vec = jnp.where(lane_iota == r, jnp.full((LANES,), bi, jnp.int32), res_vec)

    res_ref[...] = res_vec
    pltpu.sync_copy(res_ref, out_hbm.at[wid])


@functools.partial(jax.jit, static_argnames=())
def _sc_argmax(logits):
    mesh = plsc.VectorSubcoreMesh(core_axis_name="c", subcore_axis_name="s")
    f = pl.kernel(
        _body,
        out_type=jax.ShapeDtypeStruct((NUM_WORKERS, LANES), jnp.int32),
        mesh=mesh,
        scratch_types=[
            pltpu.VMEM((ROW_LEN,), jnp.float32),
            pltpu.VMEM((LANES,), jnp.int32),
        ],
    )
    return f(logits)


def kernel(logits, temperatures):
    out = _sc_argmax(logits)
    return out[:, :ROWS_PER_WORKER].reshape(NUM_ROWS)


# speedup vs baseline: 2.1742x; 2.1742x over previous
import functools
import jax
import jax.numpy as jnp
from jax import lax
from jax.experimental import pallas as pl
from jax.experimental.pallas import tpu as pltpu
from jax.experimental.pallas import tpu_sc as plsc

NUM_ROWS = 128
ROW_LEN = 100000
LANES = 16
NUM_CORES = 2
NUM_SUBCORES = 16
NUM_WORKERS = NUM_CORES * NUM_SUBCORES
ROWS_PER_WORKER = NUM_ROWS // NUM_WORKERS

ROW_VECS = ROW_LEN // LANES
UNROLL = 10
VECS_PER_ACC = ROW_VECS // UNROLL


def _merge(a, b):
    mv_a, mi_a = a
    mv_b, mi_b = b
    better = (mv_b > mv_a) | ((mv_b == mv_a) & (mi_b < mi_a))
    return (jnp.where(better, mv_b, mv_a), jnp.where(better, mi_b, mi_a))


def _body(logits_hbm, out_hbm, buf, res_ref):
    c = lax.axis_index("c")
    s = lax.axis_index("s")
    wid = c * NUM_SUBCORES + s
    row0 = wid * ROWS_PER_WORKER

    lane_iota = lax.iota(jnp.int32, LANES)
    res_vec = jnp.zeros((LANES,), jnp.int32)
    neg_inf = jnp.full((LANES,), -jnp.inf, jnp.float32)
    zeros_i = jnp.zeros((LANES,), jnp.int32)

    for r in range(ROWS_PER_WORKER):
        pltpu.sync_copy(logits_hbm.at[row0 + r], buf)

        def step(j, carry):
            new = []
            for k in range(UNROLL):
                mv, mi = carry[k]
                vn = k * VECS_PER_ACC + j
                v = buf[pl.ds(vn * LANES, LANES)]
                gt = v > mv
                mv = jnp.maximum(mv, v)
                mi = jnp.where(gt, jnp.full((LANES,), vn, jnp.int32), mi)
                new.append((mv, mi))
            return tuple(new)

        accs = lax.fori_loop(
            0,
            VECS_PER_ACC,
            step,
            tuple((neg_inf, zeros_i) for _ in range(UNROLL)),
        )

        pairs = list(accs)
        while len(pairs) > 1:
            pairs = [
                _merge(pairs[i], pairs[i + 1]) for i in range(0, len(pairs) - 1, 2)
            ] + ([pairs[-1]] if len(pairs) % 2 else [])
        m, mi = pairs[0]

        full_idx = mi * LANES + lane_iota
        bv = m[0]
        bi = full_idx[0]
        for l in range(1, LANES):
            v = m[l]
            fi = full_idx[l]
            better = (v > bv) | ((v == bv) & (fi < bi))
            bv = jnp.where(better, v, bv)
            bi = jnp.where(better, fi, bi)
        res_vec = jnp.where(lane_iota == r, jnp.full((LANES,), bi, jnp.int32), res_vec)

    res_ref[...] = res_vec
    pltpu.sync_copy(res_ref, out_hbm.at[wid])


@jax.jit
def _sc_argmax(logits):
    mesh = plsc.VectorSubcoreMesh(core_axis_name="c", subcore_axis_name="s")
    f = pl.kernel(
        _body,
        out_type=jax.ShapeDtypeStruct((NUM_WORKERS, LANES), jnp.int32),
        mesh=mesh,
        scratch_types=[
            pltpu.VMEM((ROW_LEN,), jnp.float32),
            pltpu.VMEM((LANES,), jnp.int32),
        ],
    )
    return f(logits)


def kernel(logits, temperatures):
    out = _sc_argmax(logits)
    return out[:, :ROWS_PER_WORKER].reshape(NUM_ROWS)


# speedup vs baseline: 2.2205x; 1.0213x over previous
import functools
import jax
import jax.numpy as jnp
from jax import lax
from jax.experimental import pallas as pl
from jax.experimental.pallas import tpu as pltpu
from jax.experimental.pallas import tpu_sc as plsc

NUM_ROWS = 128
ROW_LEN = 100000
LANES = 16
NUM_CORES = 2
NUM_SUBCORES = 16
NUM_WORKERS = NUM_CORES * NUM_SUBCORES
ROWS_PER_WORKER = NUM_ROWS // NUM_WORKERS

CHUNK = 19968
CHUNKS_PER_ROW = 5
SC_LEN = CHUNK * CHUNKS_PER_ROW
TAIL = ROW_LEN - SC_LEN
CHUNK_VECS = CHUNK // LANES
UNROLL = 8
STEPS = CHUNK_VECS // UNROLL
NUM_CHUNKS = ROWS_PER_WORKER * CHUNKS_PER_ROW


def _merge(a, b):
    mv_a, mi_a = a
    mv_b, mi_b = b
    better = (mv_b > mv_a) | ((mv_b == mv_a) & (mi_b < mi_a))
    return (jnp.where(better, mv_b, mv_a), jnp.where(better, mi_b, mi_a))


def _sc_body(logits_hbm, val_hbm, idx_hbm, buf0, buf1, vres_ref, ires_ref,
             sem0, sem1):
    c = lax.axis_index("c")
    s = lax.axis_index("s")
    wid = c * NUM_SUBCORES + s
    row0 = wid * ROWS_PER_WORKER

    bufs = (buf0, buf1)
    sems = (sem0, sem1)

    def start(g):
        row = row0 + g // CHUNKS_PER_ROW
        off = (g % CHUNKS_PER_ROW) * CHUNK
        return pltpu.async_copy(
            logits_hbm.at[row].at[pl.ds(off, CHUNK)], bufs[g % 2], sems[g % 2]
        )

    lane_iota = lax.iota(jnp.int32, LANES)
    vres_vec = jnp.zeros((LANES,), jnp.float32)
    ires_vec = jnp.zeros((LANES,), jnp.int32)
    neg_inf = jnp.full((LANES,), -jnp.inf, jnp.float32)
    zeros_i = jnp.zeros((LANES,), jnp.int32)

    pending = start(0)
    accs = None

    for g in range(NUM_CHUNKS):
        cbuf = bufs[g % 2]
        if g % CHUNKS_PER_ROW == 0:
            accs = tuple((neg_inf, zeros_i) for _ in range(UNROLL))
        nxt = start(g + 1) if g + 1 < NUM_CHUNKS else None
        pending.wait()
        pending = nxt

        chunk_base = (g % CHUNKS_PER_ROW) * CHUNK_VECS

        @plsc.parallel_loop(0, STEPS, carry=accs, unroll=2)
        def accs(j, carry):
            jv = jnp.full((LANES,), chunk_base + j, jnp.int32)
            new = []
            for k in range(UNROLL):
                mv, mi = carry[k]
                v = cbuf[pl.ds((k * STEPS + j) * LANES, LANES)]
                gt = v > mv
                mv = jnp.maximum(mv, v)
                mi = jnp.where(gt, jv, mi)
                new.append((mv, mi))
            return tuple(new)

        if g % CHUNKS_PER_ROW == CHUNKS_PER_ROW - 1:
            pairs = [(mv, mi + k * STEPS) for k, (mv, mi) in enumerate(accs)]
            while len(pairs) > 1:
                pairs = [
                    _merge(pairs[i], pairs[i + 1])
                    for i in range(0, len(pairs) - 1, 2)
                ] + ([pairs[-1]] if len(pairs) % 2 else [])
            m, mi = pairs[0]

            full_idx = mi * LANES + lane_iota
            bv = m[0]
            bi = full_idx[0]
            for l in range(1, LANES):
                v = m[l]
                fi = full_idx[l]
                better = (v > bv) | ((v == bv) & (fi < bi))
                bv = jnp.where(better, v, bv)
                bi = jnp.where(better, fi, bi)
            r = g // CHUNKS_PER_ROW
            sel = lane_iota == r
            vres_vec = jnp.where(sel, jnp.full((LANES,), bv, jnp.float32),
                                 vres_vec)
            ires_vec = jnp.where(sel, jnp.full((LANES,), bi, jnp.int32),
                                 ires_vec)

    vres_ref[...] = vres_vec
    ires_ref[...] = ires_vec
    pltpu.sync_copy(vres_ref, val_hbm.at[wid])
    pltpu.sync_copy(ires_ref, idx_hbm.at[wid])


def _tc_body(tail_ref, val_ref, idx_ref, out_ref):
    t = tail_ref[...]
    col = lax.broadcasted_iota(jnp.int32, (NUM_ROWS, TAIL), 1)
    tmax = jnp.max(t, axis=1, keepdims=True)
    cand = jnp.where(t == tmax, col, TAIL)
    targ = jnp.min(cand, axis=1, keepdims=True) + SC_LEN
    sc_v = val_ref[...]
    sc_i = idx_ref[...]
    out_ref[...] = jnp.where(tmax > sc_v, targ, sc_i)


@jax.jit
def _argmax_impl(logits):
    mesh = plsc.VectorSubcoreMesh(core_axis_name="c", subcore_axis_name="s")
    sc = pl.kernel(
        _sc_body,
        out_type=(
            jax.ShapeDtypeStruct((NUM_WORKERS, LANES), jnp.float32),
            jax.ShapeDtypeStruct((NUM_WORKERS, LANES), jnp.int32),
        ),
        mesh=mesh,
        scratch_types=[
            pltpu.VMEM((CHUNK,), jnp.float32),
            pltpu.VMEM((CHUNK,), jnp.float32),
            pltpu.VMEM((LANES,), jnp.float32),
            pltpu.VMEM((LANES,), jnp.int32),
            pltpu.SemaphoreType.DMA,
            pltpu.SemaphoreType.DMA,
        ],
    )
    vals, idxs = sc(logits)
    vals = vals[:, :ROWS_PER_WORKER].reshape(NUM_ROWS, 1)
    idxs = idxs[:, :ROWS_PER_WORKER].reshape(NUM_ROWS, 1)

    tail = lax.slice(logits, (0, SC_LEN), (NUM_ROWS, ROW_LEN))
    out = pl.pallas_call(
        _tc_body,
        out_shape=jax.ShapeDtypeStruct((NUM_ROWS, 1), jnp.int32),
    )(tail, vals, idxs)
    return out.reshape(NUM_ROWS)


def kernel(logits, temperatures):
    return _argmax_impl(logits)
